# baseline (speedup 1.0000x reference)
import functools

import jax
import jax.numpy as jnp
from jax import lax
from jax.experimental import pallas as pl
from jax.experimental.pallas import tpu as pltpu
from jax.experimental.pallas import tpu_sc as plsc

_NC = 2
_NS = 16
_NW = _NC * _NS
_G = 128


def _lookup_body(table_hbm, idx_hbm, out_hbm, idx_v, rows_v, gsem):
    wid = lax.axis_index("s") * _NC + lax.axis_index("c")
    bpw = idx_v.shape[0]
    base = wid * bpw
    pltpu.sync_copy(idx_hbm.at[pl.ds(base, bpw)], idx_v)
    nchunks = bpw // _G

    @pl.loop(0, nchunks)
    def _(j):
        off = j * _G
        pltpu.async_copy(table_hbm.at[idx_v.at[pl.ds(off, _G)]], rows_v, gsem).wait()
        pltpu.sync_copy(rows_v, out_hbm.at[pl.ds(base + off, _G)])


@functools.partial(jax.jit, static_argnames=())
def _gather_rows(idx_flat, table):
    b = idx_flat.shape[0]
    d = table.shape[1]
    bpw = b // _NW
    mesh = plsc.VectorSubcoreMesh(core_axis_name="c", subcore_axis_name="s")
    f = pl.kernel(
        _lookup_body,
        out_type=jax.ShapeDtypeStruct((b, d), jnp.float32),
        mesh=mesh,
        scratch_types=[
            pltpu.VMEM((bpw,), jnp.int32),
            pltpu.VMEM((_G, d), jnp.float32),
            pltpu.SemaphoreType.DMA,
        ],
        compiler_params=pltpu.CompilerParams(use_tc_tiling_on_sc=False),
    )
    return f(table, idx_flat)


def kernel(indices, table):
    b = indices.size
    out = _gather_rows(indices.reshape(b), table)
    return out.reshape(*indices.shape, table.shape[1])


# speedup vs baseline: 1.1116x; 1.1116x over previous
import functools

import jax
import jax.numpy as jnp
from jax import lax
from jax.experimental import pallas as pl
from jax.experimental.pallas import tpu as pltpu
from jax.experimental.pallas import tpu_sc as plsc

_NC = 2
_NS = 16
_NW = _NC * _NS
_G = 128
_K = 2
_S = _G * _K
_NB = 4


def _lookup_body(table_hbm, idx_hbm, out_hbm, idx_v, bufs,
                 gs0, gs1, gs2, gs3, ss0, ss1, ss2, ss3):
    gsems = (gs0, gs1, gs2, gs3)
    ssems = (ss0, ss1, ss2, ss3)
    wid = lax.axis_index("s") * _NC + lax.axis_index("c")
    bpw = idx_v.shape[0]
    base = wid * bpw
    pltpu.sync_copy(idx_hbm.at[pl.ds(base, bpw)], idx_v)
    nsc = bpw // _S

    def fire(sc, p):
        for k in range(_K):
            off = sc * _S + k * _G
            pltpu.async_copy(
                table_hbm.at[idx_v.at[pl.ds(off, _G)]],
                bufs.at[p].at[pl.ds(k * _G, _G)],
                gsems[p])

    def wait_gather(p):
        pltpu.make_async_copy(
            table_hbm.at[pl.ds(0, _S)], bufs.at[p], gsems[p]).wait()

    def wait_store(p):
        pltpu.make_async_copy(
            bufs.at[p], out_hbm.at[pl.ds(base, _S)], ssems[p]).wait()

    for p in range(_NB - 1):
        fire(p, p)

    @pl.loop(0, nsc // _NB)
    def _(g):
        for p in range(_NB):
            sc = g * _NB + p
            scf = sc + _NB - 1
            pf = (p + _NB - 1) % _NB
            if p == 0:
                @pl.when(g >= 1)
                def _():
                    wait_store(pf)
                fire(scf, pf)
            else:
                wait_store(pf)

                @pl.when(scf < nsc)
                def _():
                    fire(scf, pf)
            wait_gather(p)
            pltpu.async_copy(
                bufs.at[p], out_hbm.at[pl.ds(base + sc * _S, _S)], ssems[p])

    wait_store((nsc - 1) % _NB)


@functools.partial(jax.jit, static_argnames=())
def _gather_rows(idx_flat, table):
    b = idx_flat.shape[0]
    d = table.shape[1]
    bpw = b // _NW
    mesh = plsc.VectorSubcoreMesh(core_axis_name="c", subcore_axis_name="s")
    f = pl.kernel(
        _lookup_body,
        out_type=jax.ShapeDtypeStruct((b, d), jnp.float32),
        mesh=mesh,
        scratch_types=[
            pltpu.VMEM((bpw,), jnp.int32),
            pltpu.VMEM((_NB, _S, d), jnp.float32),
            pltpu.SemaphoreType.DMA,
            pltpu.SemaphoreType.DMA,
            pltpu.SemaphoreType.DMA,
            pltpu.SemaphoreType.DMA,
            pltpu.SemaphoreType.DMA,
            pltpu.SemaphoreType.DMA,
            pltpu.SemaphoreType.DMA,
            pltpu.SemaphoreType.DMA,
        ],
        compiler_params=pltpu.CompilerParams(use_tc_tiling_on_sc=False),
    )
    return f(table, idx_flat)


def kernel(indices, table):
    b = indices.size
    out = _gather_rows(indices.reshape(b), table)
    return out.reshape(*indices.shape, table.shape[1])
